# baseline (speedup 1.0000x reference)
import functools

import jax
import jax.numpy as jnp
from jax import lax
from jax.experimental import pallas as pl
from jax.experimental.pallas import tpu as pltpu
from jax.experimental.pallas import tpu_sc as plsc

N_TILE = 512
SC_B = 1

_NC, _NS, _NL = 2, 16, 16
_NW = _NC * _NS


def _tc_body(x1_ref, x2t_ref, dist1_ref, dist2_ref):
    i = pl.program_id(1)
    x1 = x1_ref[0]
    x2t = x2t_ref[0]

    sq1 = jnp.sum(x1 * x1, axis=1, keepdims=True)
    sq1_hi = sq1.astype(jnp.bfloat16)
    sq1_lo = (sq1 - sq1_hi.astype(jnp.float32)).astype(jnp.bfloat16)
    n = x1.shape[0]
    lhs = jnp.concatenate(
        [
            (-2.0 * x1).astype(jnp.bfloat16),
            sq1_hi,
            sq1_lo,
            jnp.ones((n, 2), jnp.bfloat16),
            jnp.zeros((n, 1), jnp.bfloat16),
        ],
        axis=1,
    )

    sq2 = jnp.sum(x2t * x2t, axis=0, keepdims=True)
    sq2_hi = sq2.astype(jnp.bfloat16)
    sq2_lo = (sq2 - sq2_hi.astype(jnp.float32)).astype(jnp.bfloat16)
    m = x2t.shape[1]
    rhs = jnp.concatenate(
        [
            x2t.astype(jnp.bfloat16),
            jnp.ones((2, m), jnp.bfloat16),
            sq2_hi,
            sq2_lo,
            jnp.zeros((1, m), jnp.bfloat16),
        ],
        axis=0,
    )

    d = jax.lax.dot_general(
        lhs, rhs, (((1,), (0,)), ((), ())),
        preferred_element_type=jnp.float32,
    )

    dist1_ref[0, 0] = jnp.maximum(jnp.min(d, axis=1), 0.0)
    part2 = jnp.maximum(jnp.min(d, axis=0), 0.0)

    @pl.when(i == 0)
    def _init():
        dist2_ref[0, 0] = part2

    @pl.when(i > 0)
    def _acc():
        dist2_ref[0, 0] = jnp.minimum(dist2_ref[0, 0], part2)


def _tc_chamfer(x1, x2t):
    B, N, D = x1.shape
    M = x2t.shape[2]
    NT = N // N_TILE
    dist1, dist2 = pl.pallas_call(
        _tc_body,
        grid=(B, NT),
        in_specs=[
            pl.BlockSpec((1, N_TILE, D), lambda b, i: (b, i, 0)),
            pl.BlockSpec((1, D, M), lambda b, i: (b, 0, 0)),
        ],
        out_specs=[
            pl.BlockSpec((1, 1, N_TILE), lambda b, i: (b * NT + i, 0, 0)),
            pl.BlockSpec((1, 1, M), lambda b, i: (b, 0, 0)),
        ],
        out_shape=[
            jax.ShapeDtypeStruct((B * NT, 1, N_TILE), jnp.float32),
            jax.ShapeDtypeStruct((B, 1, M), jnp.float32),
        ],
        compiler_params=pltpu.CompilerParams(
            dimension_semantics=("arbitrary", "arbitrary"),
        ),
    )(x1, x2t)
    return dist1.reshape(B, N), dist2.reshape(B, M)


def _rne_bf16(x):
    u = lax.bitcast_convert_type(x, jnp.uint32)
    lsb = (u >> jnp.uint32(16)) & jnp.uint32(1)
    r = (u + jnp.uint32(0x7FFF) + lsb) & jnp.uint32(0xFFFF0000)
    return lax.bitcast_convert_type(r, jnp.float32)


def _sc_build_rounded(raw_ref, ax, ay, az, asq, npts):
    def body(j, _):
        s = pl.ds(j * _NL, _NL)
        x = raw_ref[0, s]
        y = raw_ref[1, s]
        z = raw_ref[2, s]
        ax[s] = _rne_bf16(x) * 2.0
        ay[s] = _rne_bf16(y) * 2.0
        az[s] = _rne_bf16(z) * 2.0
        asq[s] = x * x + y * y + z * z
        return 0

    lax.fori_loop(0, npts // _NL, body, 0)


def _sc_direction(qx_a, qy_a, qz_a, qs_a, rx_a, ry_a, rz_a, rs_a,
                  out_v, out_hbm, b, base, nref):
    idxs = [jnp.full((_NL,), k, jnp.int32) for k in range(_NL)]
    lane = lax.iota(jnp.int32, _NL)
    rot_consts = [(lane + s) & (_NL - 1) for s in (8, 4, 2, 1)]
    nblk = nref // _NL
    ngrp = out_v.shape[0] // _NL
    big = jnp.full((_NL,), 3.0e38, jnp.float32)

    def qgroup(g, _):
        qs = pl.ds(base + g * _NL, _NL)
        qbx = qx_a[qs] * 0.5
        qby = qy_a[qs] * 0.5
        qbz = qz_a[qs] * 0.5
        res = big
        for half in range(2):
            qvec = [
                (qbx[idxs[half * 8 + k]],
                 qby[idxs[half * 8 + k]],
                 qbz[idxs[half * 8 + k]])
                for k in range(8)
            ]

            def blk_iter(jb, carry):
                for u in range(2):
                    s = pl.ds((jb * 2 + u) * _NL, _NL)
                    bx = rx_a[s]
                    by = ry_a[s]
                    bz = rz_a[s]
                    bs = rs_a[s]
                    carry = tuple(
                        jnp.minimum(
                            carry[k],
                            bs - bx * qvec[k][0] - by * qvec[k][1] - bz * qvec[k][2],
                        )
                        for k in range(8)
                    )
                return carry

            ms = lax.fori_loop(0, nblk // 2, blk_iter, (big,) * 8)
            for k in range(8):
                m = ms[k]
                for rv in rot_consts:
                    m = jnp.minimum(m, m[rv])
                res = jnp.where(lane == (half * 8 + k), m, res)
        out_v[pl.ds(g * _NL, _NL)] = jnp.maximum(res + qs_a[qs], 0.0)
        return 0

    lax.fori_loop(0, ngrp, qgroup, 0)
    pltpu.sync_copy(out_v, out_hbm.at[b, pl.ds(base, out_v.shape[0])])


def _sc_kernel_body(x1t_hbm, x2t_hbm, d1_hbm, d2_hbm,
                    x1v, x2v,
                    a1x, a1y, a1z, a1s,
                    a2x, a2y, a2z, a2s,
                    o1v, o2v):
    wid = lax.axis_index("s") * _NC + lax.axis_index("c")
    N = x1t_hbm.shape[2]
    M = x2t_hbm.shape[2]
    qpw1 = N // _NW
    qpw2 = M // _NW
    for b in range(SC_B):
        pltpu.sync_copy(x1t_hbm.at[b], x1v)
        pltpu.sync_copy(x2t_hbm.at[b], x2v)
        _sc_build_rounded(x1v, a1x, a1y, a1z, a1s, N)
        _sc_build_rounded(x2v, a2x, a2y, a2z, a2s, M)
        _sc_direction(a1x, a1y, a1z, a1s, a2x, a2y, a2z, a2s,
                      o1v, d1_hbm, b, wid * qpw1, M)
        _sc_direction(a2x, a2y, a2z, a2s, a1x, a1y, a1z, a1s,
                      o2v, d2_hbm, b, wid * qpw2, N)


def _sc_chamfer(x1t, x2t):
    SB, D, N = x1t.shape
    M = x2t.shape[2]
    mesh = plsc.VectorSubcoreMesh(core_axis_name="c", subcore_axis_name="s")
    run = pl.kernel(
        _sc_kernel_body,
        out_type=[
            jax.ShapeDtypeStruct((SB, N), jnp.float32),
            jax.ShapeDtypeStruct((SB, M), jnp.float32),
        ],
        mesh=mesh,
        scratch_types=[
            pltpu.VMEM((D, N), jnp.float32),
            pltpu.VMEM((D, M), jnp.float32),
            pltpu.VMEM((N,), jnp.float32),
            pltpu.VMEM((N,), jnp.float32),
            pltpu.VMEM((N,), jnp.float32),
            pltpu.VMEM((N,), jnp.float32),
            pltpu.VMEM((M,), jnp.float32),
            pltpu.VMEM((M,), jnp.float32),
            pltpu.VMEM((M,), jnp.float32),
            pltpu.VMEM((M,), jnp.float32),
            pltpu.VMEM((N // _NW,), jnp.float32),
            pltpu.VMEM((M // _NW,), jnp.float32),
        ],
    )
    return run(x1t, x2t)


@jax.jit
def kernel(input1, input2):
    B, N, D = input1.shape
    M = input2.shape[1]
    bt = B - SC_B
    x2t = input2.transpose(0, 2, 1)
    if SC_B:
        x1t_sc = input1[bt:].transpose(0, 2, 1)
        d1_sc, d2_sc = _sc_chamfer(x1t_sc, x2t[bt:])
    d1_tc, d2_tc = _tc_chamfer(input1[:bt], x2t[:bt])
    if SC_B:
        dist1 = jnp.concatenate([d1_tc, d1_sc], axis=0)
        dist2 = jnp.concatenate([d2_tc, d2_sc], axis=0)
    else:
        dist1, dist2 = d1_tc, d2_tc
    return (dist1, dist2)


# speedup vs baseline: 1.0016x; 1.0016x over previous
import functools

import jax
import jax.numpy as jnp
from jax import lax
from jax.experimental import pallas as pl
from jax.experimental.pallas import tpu as pltpu
from jax.experimental.pallas import tpu_sc as plsc

N_TILE = 512
SC_B = 1

_NC, _NS, _NL = 2, 16, 16
_NW = _NC * _NS


def _tc_body(x1_ref, x2t_ref, dist1_ref, dist2_ref):
    i = pl.program_id(1)
    x1 = x1_ref[0]
    x2t = x2t_ref[0]

    sq1 = jnp.sum(x1 * x1, axis=1, keepdims=True)
    sq1_hi = sq1.astype(jnp.bfloat16)
    sq1_lo = (sq1 - sq1_hi.astype(jnp.float32)).astype(jnp.bfloat16)
    n = x1.shape[0]
    lhs = jnp.concatenate(
        [
            (-2.0 * x1).astype(jnp.bfloat16),
            sq1_hi,
            sq1_lo,
            jnp.ones((n, 2), jnp.bfloat16),
            jnp.zeros((n, 1), jnp.bfloat16),
        ],
        axis=1,
    )

    sq2 = jnp.sum(x2t * x2t, axis=0, keepdims=True)
    sq2_hi = sq2.astype(jnp.bfloat16)
    sq2_lo = (sq2 - sq2_hi.astype(jnp.float32)).astype(jnp.bfloat16)
    m = x2t.shape[1]
    rhs = jnp.concatenate(
        [
            x2t.astype(jnp.bfloat16),
            jnp.ones((2, m), jnp.bfloat16),
            sq2_hi,
            sq2_lo,
            jnp.zeros((1, m), jnp.bfloat16),
        ],
        axis=0,
    )

    d = jax.lax.dot_general(
        lhs, rhs, (((1,), (0,)), ((), ())),
        preferred_element_type=jnp.float32,
    )

    dist1_ref[0, 0] = jnp.maximum(jnp.min(d, axis=1), 0.0)
    part2 = jnp.maximum(jnp.min(d, axis=0), 0.0)

    @pl.when(i == 0)
    def _init():
        dist2_ref[0, 0] = part2

    @pl.when(i > 0)
    def _acc():
        dist2_ref[0, 0] = jnp.minimum(dist2_ref[0, 0], part2)


def _tc_chamfer(x1, x2t):
    B, N, D = x1.shape
    M = x2t.shape[2]
    NT = N // N_TILE
    dist1, dist2 = pl.pallas_call(
        _tc_body,
        grid=(B, NT),
        in_specs=[
            pl.BlockSpec((1, N_TILE, D), lambda b, i: (b, i, 0)),
            pl.BlockSpec((1, D, M), lambda b, i: (b, 0, 0)),
        ],
        out_specs=[
            pl.BlockSpec((1, 1, N_TILE), lambda b, i: (b * NT + i, 0, 0)),
            pl.BlockSpec((1, 1, M), lambda b, i: (b, 0, 0)),
        ],
        out_shape=[
            jax.ShapeDtypeStruct((B * NT, 1, N_TILE), jnp.float32),
            jax.ShapeDtypeStruct((B, 1, M), jnp.float32),
        ],
        compiler_params=pltpu.CompilerParams(
            dimension_semantics=("arbitrary", "arbitrary"),
        ),
    )(x1, x2t)
    return dist1.reshape(B, N), dist2.reshape(B, M)


def _rne_bf16(x):
    u = lax.bitcast_convert_type(x, jnp.uint32)
    lsb = (u >> jnp.uint32(16)) & jnp.uint32(1)
    r = (u + jnp.uint32(0x7FFF) + lsb) & jnp.uint32(0xFFFF0000)
    return lax.bitcast_convert_type(r, jnp.float32)


def _sc_build_rounded(raw_ref, ax, ay, az, asq, npts):
    def body(j, _):
        s = pl.ds(j * _NL, _NL)
        x = raw_ref[0, s]
        y = raw_ref[1, s]
        z = raw_ref[2, s]
        ax[s] = _rne_bf16(x) * 2.0
        ay[s] = _rne_bf16(y) * 2.0
        az[s] = _rne_bf16(z) * 2.0
        asq[s] = x * x + y * y + z * z
        return 0

    lax.fori_loop(0, npts // _NL, body, 0)


def _sc_direction(qx_a, qy_a, qz_a, qs_a, rx_a, ry_a, rz_a, rs_a,
                  out_v, out_hbm, b, base, nref):
    idxs = [jnp.full((_NL,), k, jnp.int32) for k in range(_NL)]
    lane = lax.iota(jnp.int32, _NL)
    rot_consts = [(lane + s) & (_NL - 1) for s in (8, 4, 2, 1)]
    nblk = nref // _NL
    ngrp = out_v.shape[0] // _NL
    big = jnp.full((_NL,), 3.0e38, jnp.float32)

    def qgroup(g, _):
        qs = pl.ds(base + g * _NL, _NL)
        qbx = qx_a[qs] * 0.5
        qby = qy_a[qs] * 0.5
        qbz = qz_a[qs] * 0.5
        res = big
        for half in range(2):
            qvec = [
                (qbx[idxs[half * 8 + k]],
                 qby[idxs[half * 8 + k]],
                 qbz[idxs[half * 8 + k]])
                for k in range(8)
            ]

            @plsc.parallel_loop(0, nblk, step=1, unroll=2, carry=(big,) * 8)
            def ms(jb, carry):
                s = pl.ds(jb * _NL, _NL)
                bx = rx_a[s]
                by = ry_a[s]
                bz = rz_a[s]
                bs = rs_a[s]
                return tuple(
                    jnp.minimum(
                        carry[k],
                        bs - bx * qvec[k][0] - by * qvec[k][1] - bz * qvec[k][2],
                    )
                    for k in range(8)
                )
            for k in range(8):
                m = ms[k]
                for rv in rot_consts:
                    m = jnp.minimum(m, m[rv])
                res = jnp.where(lane == (half * 8 + k), m, res)
        out_v[pl.ds(g * _NL, _NL)] = jnp.maximum(res + qs_a[qs], 0.0)
        return 0

    lax.fori_loop(0, ngrp, qgroup, 0)
    pltpu.sync_copy(out_v, out_hbm.at[b, pl.ds(base, out_v.shape[0])])


def _sc_kernel_body(x1t_hbm, x2t_hbm, d1_hbm, d2_hbm,
                    x1v, x2v,
                    a1x, a1y, a1z, a1s,
                    a2x, a2y, a2z, a2s,
                    o1v, o2v):
    wid = lax.axis_index("s") * _NC + lax.axis_index("c")
    N = x1t_hbm.shape[2]
    M = x2t_hbm.shape[2]
    qpw1 = N // _NW
    qpw2 = M // _NW
    for b in range(SC_B):
        pltpu.sync_copy(x1t_hbm.at[b], x1v)
        pltpu.sync_copy(x2t_hbm.at[b], x2v)
        _sc_build_rounded(x1v, a1x, a1y, a1z, a1s, N)
        _sc_build_rounded(x2v, a2x, a2y, a2z, a2s, M)
        _sc_direction(a1x, a1y, a1z, a1s, a2x, a2y, a2z, a2s,
                      o1v, d1_hbm, b, wid * qpw1, M)
        _sc_direction(a2x, a2y, a2z, a2s, a1x, a1y, a1z, a1s,
                      o2v, d2_hbm, b, wid * qpw2, N)


def _sc_chamfer(x1t, x2t):
    SB, D, N = x1t.shape
    M = x2t.shape[2]
    mesh = plsc.VectorSubcoreMesh(core_axis_name="c", subcore_axis_name="s")
    run = pl.kernel(
        _sc_kernel_body,
        out_type=[
            jax.ShapeDtypeStruct((SB, N), jnp.float32),
            jax.ShapeDtypeStruct((SB, M), jnp.float32),
        ],
        mesh=mesh,
        scratch_types=[
            pltpu.VMEM((D, N), jnp.float32),
            pltpu.VMEM((D, M), jnp.float32),
            pltpu.VMEM((N,), jnp.float32),
            pltpu.VMEM((N,), jnp.float32),
            pltpu.VMEM((N,), jnp.float32),
            pltpu.VMEM((N,), jnp.float32),
            pltpu.VMEM((M,), jnp.float32),
            pltpu.VMEM((M,), jnp.float32),
            pltpu.VMEM((M,), jnp.float32),
            pltpu.VMEM((M,), jnp.float32),
            pltpu.VMEM((N // _NW,), jnp.float32),
            pltpu.VMEM((M // _NW,), jnp.float32),
        ],
    )
    return run(x1t, x2t)


@jax.jit
def kernel(input1, input2):
    B, N, D = input1.shape
    M = input2.shape[1]
    bt = B - SC_B
    x2t = input2.transpose(0, 2, 1)
    if SC_B:
        x1t_sc = input1[bt:].transpose(0, 2, 1)
        d1_sc, d2_sc = _sc_chamfer(x1t_sc, x2t[bt:])
    d1_tc, d2_tc = _tc_chamfer(input1[:bt], x2t[:bt])
    if SC_B:
        dist1 = jnp.concatenate([d1_tc, d1_sc], axis=0)
        dist2 = jnp.concatenate([d2_tc, d2_sc], axis=0)
    else:
        dist1, dist2 = d1_tc, d2_tc
    return (dist1, dist2)
